# Initial kernel scaffold; baseline (speedup 1.0000x reference)
import jax
import jax.numpy as jnp
from jax.experimental import pallas as pl


def kernel(x_sv, x_trk, batch_sv, batch_trk, sv_w1, sv_b1, sv_w2, sv_b2, trk_w1, trk_b1, trk_w2, trk_b2, c1_w, c1_b, c2_w, c2_b, o_w1, o_b1, o_w2, o_b2, o_w3, o_b3, o_w4, o_b4):
    raise NotImplementedError("write your pallas kernel here")


# speedup vs baseline: 7.9629x; 7.9629x over previous
import functools

import jax
import jax.numpy as jnp
from jax import lax
from jax.experimental import pallas as pl
from jax.experimental.pallas import tpu as pltpu
from jax.experimental.pallas import tpu_sc as plsc

_HID = 128
_K = 8
_B = 128
_N_SV = 4096
_N_TRK = 8192
_TILE = 128
_BIG = 1e30
_INF = 1e38


def _elu(x):
    return jnp.where(x > 0, x, jnp.exp(jnp.minimum(x, 0.0)) - 1.0)


def _enc_sv_body(x_ref, w1_ref, b1_ref, w2_ref, b2_ref, ybot_ref,
                 enc_ref, y1_ref, n_ref):
    x = x_ref[...]
    h = _elu(jnp.dot(x, w1_ref[...], preferred_element_type=jnp.float32)
             + b1_ref[...])
    enc = jax.nn.relu(jnp.dot(h, w2_ref[...],
                              preferred_element_type=jnp.float32) + b2_ref[...])
    enc_ref[...] = enc
    y1_ref[...] = jnp.dot(enc, ybot_ref[...], preferred_element_type=jnp.float32)
    n_ref[...] = jnp.sum(enc * enc, axis=1, keepdims=True)


def _enc_trk_body(x_ref, w1_ref, b1_ref, w2_ref, b2_ref,
                  wd1_ref, cb1_ref, wd2_ref, cb2_ref,
                  enc_ref, a1_ref, a2_ref, n_ref):
    x = x_ref[...]
    h = _elu(jnp.dot(x, w1_ref[...], preferred_element_type=jnp.float32)
             + b1_ref[...])
    enc = jax.nn.relu(jnp.dot(h, w2_ref[...],
                              preferred_element_type=jnp.float32) + b2_ref[...])
    enc_ref[...] = enc
    a1_ref[...] = jnp.dot(enc, wd1_ref[...],
                          preferred_element_type=jnp.float32) + cb1_ref[...]
    a2_ref[...] = jnp.dot(enc, wd2_ref[...],
                          preferred_element_type=jnp.float32) + cb2_ref[...]
    n_ref[...] = jnp.sum(enc * enc, axis=1, keepdims=True)


def _feats1_body(a1_ref, m1_ref, ybot_ref, f1_ref, y2_ref, n_ref):
    f1 = _elu(a1_ref[...] + m1_ref[...])
    f1_ref[...] = f1
    y2_ref[...] = jnp.dot(f1, ybot_ref[...], preferred_element_type=jnp.float32)
    n_ref[...] = jnp.sum(f1 * f1, axis=1, keepdims=True)


def _knn_body(sband_ref, bdst_ref, dst_ref, dstn_ref,
              src_ref, srcn_ref, bsrc_ref, idx_ref):
    t = pl.program_id(0)
    dstt = dst_ref[...]
    dn = dstn_ref[...]
    bd = bdst_ref[...]
    col8 = lax.broadcasted_iota(jnp.int32, (_TILE, _K), 1)
    colb = lax.broadcasted_iota(jnp.int32, (_TILE, _TILE), 1)

    def merge(blk, rv, ri):
        sb = src_ref[pl.ds(blk * _TILE, _TILE), :]
        sn = srcn_ref[pl.ds(blk, 1), :]
        bs = bsrc_ref[pl.ds(blk, 1), :]
        dot = lax.dot_general(dstt, sb, (((1,), (1,)), ((), ())),
                              preferred_element_type=jnp.float32)
        d2 = (dn - 2.0 * dot) + sn
        d2 = jnp.where(bd == bs, d2, _BIG)
        gbase = blk * _TILE
        nv = []
        ni = []
        for _ in range(_K):
            m1 = jnp.min(rv, axis=1, keepdims=True)
            m2 = jnp.min(d2, axis=1, keepdims=True)
            c1 = jnp.min(jnp.where(rv == m1, col8, 2**30), axis=1, keepdims=True)
            c2 = jnp.min(jnp.where(d2 == m2, colb, 2**30), axis=1, keepdims=True)
            i1 = jnp.sum(jnp.where(col8 == c1, ri, 0), axis=1, keepdims=True)
            take1 = m1 <= m2
            nv.append(jnp.where(take1, m1, m2))
            ni.append(jnp.where(take1, i1, gbase + c2))
            rv = jnp.where(take1 & (col8 == c1), _INF, rv)
            d2 = jnp.where(jnp.logical_not(take1) & (colb == c2), _INF, d2)
        rv = sum(jnp.where(col8 == k, nv[k], 0.0) for k in range(_K))
        ri = sum(jnp.where(col8 == k, ni[k], 0) for k in range(_K))
        return rv, ri

    rv0 = jnp.full((_TILE, _K), _INF, jnp.float32)
    ri0 = jnp.zeros((_TILE, _K), jnp.int32)
    rv0, ri0 = merge(0, rv0, ri0)
    blo = jnp.maximum(sband_ref[t, 0], 1)
    bhi = sband_ref[t, 1]

    def body(b, carry):
        return merge(b, carry[0], carry[1])

    rv0, ri0 = lax.fori_loop(blo, bhi, body, (rv0, ri0))
    idx_ref[...] = ri0


def _banded_knn(sband, batch_dst, dst, dstn, src, srcn_rows, bsrc_rows):
    n_dst = dst.shape[0]
    n_src = src.shape[0]
    nblk = n_src // _TILE
    grid = (n_dst // _TILE,)
    return pl.pallas_call(
        _knn_body,
        grid_spec=pltpu.PrefetchScalarGridSpec(
            num_scalar_prefetch=1,
            grid=grid,
            in_specs=[
                pl.BlockSpec((_TILE, 1), lambda t, s: (t, 0)),
                pl.BlockSpec((_TILE, _HID), lambda t, s: (t, 0)),
                pl.BlockSpec((_TILE, 1), lambda t, s: (t, 0)),
                pl.BlockSpec((n_src, _HID), lambda t, s: (0, 0)),
                pl.BlockSpec((nblk, _TILE), lambda t, s: (0, 0)),
                pl.BlockSpec((nblk, _TILE), lambda t, s: (0, 0)),
            ],
            out_specs=pl.BlockSpec((_TILE, _K), lambda t, s: (t, 0)),
        ),
        out_shape=jax.ShapeDtypeStruct((n_dst, _K), jnp.int32),
    )(sband, batch_dst, dst, dstn, src, srcn_rows, bsrc_rows)


def _band_bounds(batch_dst, batch_src):
    bd = batch_dst.reshape(-1, _TILE)
    lo = jnp.searchsorted(batch_src, bd[:, 0], side='left').astype(jnp.int32)
    hi = jnp.searchsorted(batch_src, bd[:, -1], side='right').astype(jnp.int32)
    return jnp.stack([lo // _TILE, (hi + _TILE - 1) // _TILE], axis=1)


_SC_CHUNK = 16
_SC_WORKERS = 32
_SC_PER_W = _N_TRK // _SC_WORKERS


def _gather_max(y, idx_flat):
    mesh = plsc.VectorSubcoreMesh(core_axis_name="c", subcore_axis_name="s")

    @functools.partial(
        pl.kernel,
        out_type=jax.ShapeDtypeStruct((_N_TRK, _HID), jnp.float32),
        mesh=mesh,
        scratch_types=[
            pltpu.VMEM((_SC_CHUNK * _K,), jnp.int32),
            pltpu.VMEM((_SC_CHUNK * _K, _HID), jnp.float32),
            pltpu.VMEM((_SC_CHUNK, _HID), jnp.float32),
            pltpu.SemaphoreType.DMA,
        ],
    )
    def k(y_h, idx_h, out_h, idx_v, rows_v, out_v, sem):
        wid = lax.axis_index("s") * 2 + lax.axis_index("c")

        def chunk(ci, _):
            base = wid * _SC_PER_W + ci * _SC_CHUNK
            pltpu.sync_copy(idx_h.at[pl.ds(base * _K, _SC_CHUNK * _K)], idx_v)
            pltpu.async_copy(y_h.at[idx_v], rows_v, sem).wait()

            def per_t(t, _):
                r0 = t * _K
                for c in range(_HID // 16):
                    acc = rows_v[r0, pl.ds(c * 16, 16)]
                    for j in range(1, _K):
                        acc = jnp.maximum(acc, rows_v[r0 + j, pl.ds(c * 16, 16)])
                    out_v[t, pl.ds(c * 16, 16)] = acc
                return 0

            lax.fori_loop(0, _SC_CHUNK, per_t, 0)
            pltpu.sync_copy(out_v, out_h.at[pl.ds(base, _SC_CHUNK)])
            return 0

        lax.fori_loop(0, _SC_PER_W // _SC_CHUNK, chunk, 0)

    return k(y, idx_flat)


_POOL_BLK = 512


def _pool_body(a2_ref, m2_ref, bt_ref,
               w1_ref, b1_ref, w2_ref, b2_ref, w3_ref, b3_ref, w4_ref, b4_ref,
               out_ref, acc_ref, cnt_ref):
    i = pl.program_id(0)

    @pl.when(i == 0)
    def _():
        acc_ref[...] = jnp.zeros_like(acc_ref)
        cnt_ref[...] = jnp.zeros_like(cnt_ref)

    f2 = _elu(a2_ref[...] + m2_ref[...])
    seg = lax.broadcasted_iota(jnp.int32, (1, _B), 1)
    oh = (bt_ref[...] == seg).astype(jnp.float32)
    acc_ref[...] += lax.dot_general(oh, f2, (((0,), (0,)), ((), ())),
                                    preferred_element_type=jnp.float32)
    cnt_ref[...] += jnp.sum(oh, axis=0, keepdims=True)

    @pl.when(i == pl.num_programs(0) - 1)
    def _():
        cnt = jnp.maximum(cnt_ref[...], 1.0)
        pooled = acc_ref[...] / cnt.reshape(_B, 1)
        h = _elu(jnp.dot(pooled, w1_ref[...],
                         preferred_element_type=jnp.float32) + b1_ref[...])
        h = _elu(jnp.dot(h, w2_ref[...],
                         preferred_element_type=jnp.float32) + b2_ref[...])
        h = _elu(jnp.dot(h, w3_ref[...],
                         preferred_element_type=jnp.float32) + b3_ref[...])
        out_ref[...] = jnp.dot(h, w4_ref[...],
                               preferred_element_type=jnp.float32) + b4_ref[...]


def kernel(x_sv, x_trk, batch_sv, batch_trk,
           sv_w1, sv_b1, sv_w2, sv_b2, trk_w1, trk_b1, trk_w2, trk_b2,
           c1_w, c1_b, c2_w, c2_b,
           o_w1, o_b1, o_w2, o_b2, o_w3, o_b3, o_w4, o_b4):
    c1_top, c1_bot = c1_w[:_HID], c1_w[_HID:]
    c2_top, c2_bot = c2_w[:_HID], c2_w[_HID:]
    wd1 = c1_top - c1_bot
    wd2 = c2_top - c2_bot

    row = lambda v: v.reshape(1, -1)

    sv_enc, y1, sv_n = pl.pallas_call(
        _enc_sv_body,
        grid=(_N_SV // _POOL_BLK,),
        in_specs=[
            pl.BlockSpec((_POOL_BLK, 2), lambda i: (i, 0)),
            pl.BlockSpec((2, _HID), lambda i: (0, 0)),
            pl.BlockSpec((1, _HID), lambda i: (0, 0)),
            pl.BlockSpec((_HID, _HID), lambda i: (0, 0)),
            pl.BlockSpec((1, _HID), lambda i: (0, 0)),
            pl.BlockSpec((_HID, _HID), lambda i: (0, 0)),
        ],
        out_specs=[
            pl.BlockSpec((_POOL_BLK, _HID), lambda i: (i, 0)),
            pl.BlockSpec((_POOL_BLK, _HID), lambda i: (i, 0)),
            pl.BlockSpec((_POOL_BLK, 1), lambda i: (i, 0)),
        ],
        out_shape=[
            jax.ShapeDtypeStruct((_N_SV, _HID), jnp.float32),
            jax.ShapeDtypeStruct((_N_SV, _HID), jnp.float32),
            jax.ShapeDtypeStruct((_N_SV, 1), jnp.float32),
        ],
    )(x_sv, sv_w1, row(sv_b1), sv_w2, row(sv_b2), c1_bot)

    trk_enc, a1, a2, trk_n = pl.pallas_call(
        _enc_trk_body,
        grid=(_N_TRK // _POOL_BLK,),
        in_specs=[
            pl.BlockSpec((_POOL_BLK, 8), lambda i: (i, 0)),
            pl.BlockSpec((8, _HID), lambda i: (0, 0)),
            pl.BlockSpec((1, _HID), lambda i: (0, 0)),
            pl.BlockSpec((_HID, _HID), lambda i: (0, 0)),
            pl.BlockSpec((1, _HID), lambda i: (0, 0)),
            pl.BlockSpec((_HID, _HID), lambda i: (0, 0)),
            pl.BlockSpec((1, _HID), lambda i: (0, 0)),
            pl.BlockSpec((_HID, _HID), lambda i: (0, 0)),
            pl.BlockSpec((1, _HID), lambda i: (0, 0)),
        ],
        out_specs=[
            pl.BlockSpec((_POOL_BLK, _HID), lambda i: (i, 0)),
            pl.BlockSpec((_POOL_BLK, _HID), lambda i: (i, 0)),
            pl.BlockSpec((_POOL_BLK, _HID), lambda i: (i, 0)),
            pl.BlockSpec((_POOL_BLK, 1), lambda i: (i, 0)),
        ],
        out_shape=[
            jax.ShapeDtypeStruct((_N_TRK, _HID), jnp.float32),
            jax.ShapeDtypeStruct((_N_TRK, _HID), jnp.float32),
            jax.ShapeDtypeStruct((_N_TRK, _HID), jnp.float32),
            jax.ShapeDtypeStruct((_N_TRK, 1), jnp.float32),
        ],
    )(x_trk, trk_w1, row(trk_b1), trk_w2, row(trk_b2),
      wd1, row(c1_b), wd2, row(c2_b))

    bsv_rows = batch_sv.reshape(-1, _TILE)
    btrk_rows = batch_trk.reshape(-1, _TILE)
    bsv_col = batch_sv.reshape(-1, 1)
    btrk_col = batch_trk.reshape(-1, 1)

    sband1 = _band_bounds(batch_trk, batch_sv)
    idx1 = _banded_knn(sband1, btrk_col, trk_enc, trk_n,
                       sv_enc, sv_n.reshape(-1, _TILE), bsv_rows)
    m1 = _gather_max(y1, idx1.reshape(-1))

    f1, y2, f1_n = pl.pallas_call(
        _feats1_body,
        grid=(_N_TRK // _POOL_BLK,),
        in_specs=[
            pl.BlockSpec((_POOL_BLK, _HID), lambda i: (i, 0)),
            pl.BlockSpec((_POOL_BLK, _HID), lambda i: (i, 0)),
            pl.BlockSpec((_HID, _HID), lambda i: (0, 0)),
        ],
        out_specs=[
            pl.BlockSpec((_POOL_BLK, _HID), lambda i: (i, 0)),
            pl.BlockSpec((_POOL_BLK, _HID), lambda i: (i, 0)),
            pl.BlockSpec((_POOL_BLK, 1), lambda i: (i, 0)),
        ],
        out_shape=[
            jax.ShapeDtypeStruct((_N_TRK, _HID), jnp.float32),
            jax.ShapeDtypeStruct((_N_TRK, _HID), jnp.float32),
            jax.ShapeDtypeStruct((_N_TRK, 1), jnp.float32),
        ],
    )(a1, m1, c2_bot)

    sband2 = _band_bounds(batch_trk, batch_trk)
    idx2 = _banded_knn(sband2, btrk_col, trk_enc, trk_n,
                       f1, f1_n.reshape(-1, _TILE), btrk_rows)
    m2 = _gather_max(y2, idx2.reshape(-1))

    out = pl.pallas_call(
        _pool_body,
        grid=(_N_TRK // _POOL_BLK,),
        in_specs=[
            pl.BlockSpec((_POOL_BLK, _HID), lambda i: (i, 0)),
            pl.BlockSpec((_POOL_BLK, _HID), lambda i: (i, 0)),
            pl.BlockSpec((_POOL_BLK, 1), lambda i: (i, 0)),
            pl.BlockSpec((_HID, 64), lambda i: (0, 0)),
            pl.BlockSpec((1, 64), lambda i: (0, 0)),
            pl.BlockSpec((64, 32), lambda i: (0, 0)),
            pl.BlockSpec((1, 32), lambda i: (0, 0)),
            pl.BlockSpec((32, 4), lambda i: (0, 0)),
            pl.BlockSpec((1, 4), lambda i: (0, 0)),
            pl.BlockSpec((4, 1), lambda i: (0, 0)),
            pl.BlockSpec((1, 1), lambda i: (0, 0)),
        ],
        out_specs=pl.BlockSpec((_B, 1), lambda i: (0, 0)),
        out_shape=jax.ShapeDtypeStruct((_B, 1), jnp.float32),
        scratch_shapes=[
            pltpu.VMEM((_B, _HID), jnp.float32),
            pltpu.VMEM((1, _B), jnp.float32),
        ],
    )(a2, m2, btrk_col, o_w1, row(o_b1), o_w2, row(o_b2),
      o_w3, row(o_b3), o_w4, row(o_b4))

    return (out, jnp.arange(_B, dtype=batch_trk.dtype))


# speedup vs baseline: 23.9492x; 3.0076x over previous
import functools

import jax
import jax.numpy as jnp
from jax import lax
from jax.experimental import pallas as pl
from jax.experimental.pallas import tpu as pltpu
from jax.experimental.pallas import tpu_sc as plsc

_HID = 128
_K = 8
_B = 128
_N_SV = 4096
_N_TRK = 8192
_TILE = 128
_BIG = 1e30
_INF = 1e38


def _elu(x):
    return jnp.where(x > 0, x, jnp.exp(jnp.minimum(x, 0.0)) - 1.0)


def _enc_sv_body(x_ref, w1_ref, b1_ref, w2_ref, b2_ref, ybot_ref,
                 enc_ref, y1_ref, n_ref):
    x = x_ref[...]
    h = _elu(jnp.dot(x, w1_ref[...], preferred_element_type=jnp.float32)
             + b1_ref[...])
    enc = jax.nn.relu(jnp.dot(h, w2_ref[...],
                              preferred_element_type=jnp.float32) + b2_ref[...])
    enc_ref[...] = enc
    y1_ref[...] = jnp.dot(enc, ybot_ref[...], preferred_element_type=jnp.float32)
    n_ref[...] = jnp.sum(enc * enc, axis=1, keepdims=True)


def _enc_trk_body(x_ref, w1_ref, b1_ref, w2_ref, b2_ref,
                  wd1_ref, cb1_ref, wd2_ref, cb2_ref,
                  enc_ref, a1_ref, a2_ref, n_ref):
    x = x_ref[...]
    h = _elu(jnp.dot(x, w1_ref[...], preferred_element_type=jnp.float32)
             + b1_ref[...])
    enc = jax.nn.relu(jnp.dot(h, w2_ref[...],
                              preferred_element_type=jnp.float32) + b2_ref[...])
    enc_ref[...] = enc
    a1_ref[...] = jnp.dot(enc, wd1_ref[...],
                          preferred_element_type=jnp.float32) + cb1_ref[...]
    a2_ref[...] = jnp.dot(enc, wd2_ref[...],
                          preferred_element_type=jnp.float32) + cb2_ref[...]
    n_ref[...] = jnp.sum(enc * enc, axis=1, keepdims=True)


def _feats1_body(a1_ref, m1_ref, ybot_ref, f1_ref, y2_ref, n_ref):
    f1 = _elu(a1_ref[...] + m1_ref[...])
    f1_ref[...] = f1
    y2_ref[...] = jnp.dot(f1, ybot_ref[...], preferred_element_type=jnp.float32)
    n_ref[...] = jnp.sum(f1 * f1, axis=1, keepdims=True)


_MASKH = -128
_MAXK = 2147483647


def _knn_body(sband_ref, bdst_ref, dst_ref, dstn_ref,
              src_ref, srcn_ref, bsrc_ref, idx_ref):
    t = pl.program_id(0)
    dstt = dst_ref[...]
    dn = dstn_ref[0]
    bd = bdst_ref[0]
    slot8 = lax.broadcasted_iota(jnp.int32, (_K, _TILE), 0)
    rowi = lax.broadcasted_iota(jnp.int32, (_TILE, _TILE), 0)

    def merge(blk, rv, ri):
        gbase = blk * _TILE
        sb = src_ref[pl.ds(gbase, _TILE), :]
        sn = srcn_ref[pl.ds(gbase, _TILE), :]
        bs = bsrc_ref[pl.ds(gbase, _TILE), :]
        dot = lax.dot_general(sb, dstt, (((1,), (1,)), ((), ())),
                              preferred_element_type=jnp.float32)
        d2 = jnp.maximum((sn - 2.0 * dot) + dn, 0.0)
        d2 = jnp.where(bs == bd, d2, _BIG)
        keys = (lax.bitcast_convert_type(d2, jnp.int32) & _MASKH) | rowi
        nv = []
        ni = []
        for _ in range(_K):
            m1 = jnp.min(rv, axis=0, keepdims=True)
            m2 = jnp.min(keys, axis=0, keepdims=True)
            take1 = (m1 & _MASKH) <= (m2 & _MASKH)
            i1 = jnp.min(jnp.where(rv == m1, ri, _MAXK), axis=0, keepdims=True)
            i2 = gbase + (m2 & 127)
            nv.append(jnp.where(take1, m1, m2))
            ni.append(jnp.where(take1, i1, i2))
            rv = jnp.where((rv == m1) & take1, _MAXK, rv)
            keys = jnp.where((keys == m2) & jnp.logical_not(take1), _MAXK, keys)
        newv = jnp.concatenate(nv, axis=0)
        newi = jnp.concatenate(ni, axis=0)
        return (newv & _MASKH) | slot8, newi

    def body(b, carry):
        return merge(b, carry[0], carry[1])

    blo = sband_ref[t, 0]
    bhi = sband_ref[t, 1]
    rv0 = jnp.full((_K, _TILE), 0x7E000000, jnp.int32) | slot8
    ri0 = jnp.zeros((_K, _TILE), jnp.int32)

    def with_blk0():
        c = merge(0, rv0, ri0)
        return lax.fori_loop(jnp.maximum(blo, 1), bhi, body, c)

    def without_blk0():
        return lax.fori_loop(blo, bhi, body, (rv0, ri0))

    _, ri = lax.cond(sband_ref[t, 2] == 1, with_blk0, without_blk0)
    idx_ref[...] = ri


def _banded_knn(sband, bdst_rows, dst, dstn_rows, src, srcn_col, bsrc_col):
    n_dst = dst.shape[0]
    n_src = src.shape[0]
    grid = (n_dst // _TILE,)
    return pl.pallas_call(
        _knn_body,
        grid_spec=pltpu.PrefetchScalarGridSpec(
            num_scalar_prefetch=1,
            grid=grid,
            in_specs=[
                pl.BlockSpec((1, 1, _TILE), lambda t, s: (t, 0, 0)),
                pl.BlockSpec((_TILE, _HID), lambda t, s: (t, 0)),
                pl.BlockSpec((1, 1, _TILE), lambda t, s: (t, 0, 0)),
                pl.BlockSpec((n_src, _HID), lambda t, s: (0, 0)),
                pl.BlockSpec((n_src, 1), lambda t, s: (0, 0)),
                pl.BlockSpec((n_src, 1), lambda t, s: (0, 0)),
            ],
            out_specs=pl.BlockSpec((_K, _TILE), lambda t, s: (0, t)),
        ),
        out_shape=jax.ShapeDtypeStruct((_K, n_dst), jnp.int32),
    )(sband, bdst_rows, dst, dstn_rows, src, srcn_col, bsrc_col)


def _band_bounds(batch_dst, batch_src):
    bd = batch_dst.reshape(-1, _TILE)
    b_lo = bd[:, 0]
    b_hi = bd[:, -1]
    lo = jnp.searchsorted(batch_src, b_lo, side='left').astype(jnp.int32)
    hi = jnp.searchsorted(batch_src, b_hi, side='right').astype(jnp.int32)
    segs = jnp.arange(_B, dtype=batch_src.dtype)
    cnt = (jnp.searchsorted(batch_src, segs, side='right')
           - jnp.searchsorted(batch_src, segs, side='left'))
    in_rng = (segs[None, :] >= b_lo[:, None]) & (segs[None, :] <= b_hi[:, None])
    mincnt = jnp.min(jnp.where(in_rng, cnt[None, :], 1 << 30), axis=1)
    need0 = (mincnt < _K).astype(jnp.int32)
    return jnp.stack([lo // _TILE, (hi + _TILE - 1) // _TILE, need0], axis=1)


_SC_CHUNK = 16
_SC_WORKERS = 32
_SC_PER_W = _N_TRK // _SC_WORKERS


def _gather_max(y, idx_t):
    mesh = plsc.VectorSubcoreMesh(core_axis_name="c", subcore_axis_name="s")

    @functools.partial(
        pl.kernel,
        out_type=jax.ShapeDtypeStruct((_N_TRK, _HID), jnp.float32),
        mesh=mesh,
        scratch_types=[
            pltpu.VMEM((_K, _SC_PER_W), jnp.int32),
            pltpu.VMEM((_SC_CHUNK * _K,), jnp.int32),
            pltpu.VMEM((_SC_CHUNK * _K, _HID), jnp.float32),
            pltpu.VMEM((_SC_CHUNK, _HID), jnp.float32),
            pltpu.SemaphoreType.DMA,
        ],
    )
    def k(y_h, idx_h, out_h, idx_all, idx_v, rows_v, out_v, sem):
        wid = lax.axis_index("s") * 2 + lax.axis_index("c")
        for s in range(_K):
            pltpu.sync_copy(idx_h.at[s, pl.ds(wid * _SC_PER_W, _SC_PER_W)],
                            idx_all.at[s])

        def chunk(ci, _):
            base = wid * _SC_PER_W + ci * _SC_CHUNK
            for s in range(_K):
                idx_v[pl.ds(s * _SC_CHUNK, _SC_CHUNK)] = (
                    idx_all[s, pl.ds(ci * _SC_CHUNK, _SC_CHUNK)])
            pltpu.async_copy(y_h.at[idx_v], rows_v, sem).wait()

            def per_t(t, _):
                for c in range(_HID // 16):
                    acc = rows_v[t, pl.ds(c * 16, 16)]
                    for s in range(1, _K):
                        acc = jnp.maximum(
                            acc, rows_v[s * _SC_CHUNK + t, pl.ds(c * 16, 16)])
                    out_v[t, pl.ds(c * 16, 16)] = acc
                return 0

            lax.fori_loop(0, _SC_CHUNK, per_t, 0)
            pltpu.sync_copy(out_v, out_h.at[pl.ds(base, _SC_CHUNK)])
            return 0

        lax.fori_loop(0, _SC_PER_W // _SC_CHUNK, chunk, 0)

    return k(y, idx_t)


_POOL_BLK = 512


def _pool_body(a2_ref, m2_ref, bt_ref,
               w1_ref, b1_ref, w2_ref, b2_ref, w3_ref, b3_ref, w4_ref, b4_ref,
               out_ref, acc_ref, cnt_ref):
    i = pl.program_id(0)

    @pl.when(i == 0)
    def _():
        acc_ref[...] = jnp.zeros_like(acc_ref)
        cnt_ref[...] = jnp.zeros_like(cnt_ref)

    f2 = _elu(a2_ref[...] + m2_ref[...])
    seg = lax.broadcasted_iota(jnp.int32, (1, _B), 1)
    oh = (bt_ref[...] == seg).astype(jnp.float32)
    acc_ref[...] += lax.dot_general(oh, f2, (((0,), (0,)), ((), ())),
                                    preferred_element_type=jnp.float32)
    cnt_ref[...] += jnp.sum(oh, axis=0, keepdims=True)

    @pl.when(i == pl.num_programs(0) - 1)
    def _():
        cnt = jnp.maximum(cnt_ref[...], 1.0)
        pooled = acc_ref[...] / cnt.reshape(_B, 1)
        h = _elu(jnp.dot(pooled, w1_ref[...],
                         preferred_element_type=jnp.float32) + b1_ref[...])
        h = _elu(jnp.dot(h, w2_ref[...],
                         preferred_element_type=jnp.float32) + b2_ref[...])
        h = _elu(jnp.dot(h, w3_ref[...],
                         preferred_element_type=jnp.float32) + b3_ref[...])
        out_ref[...] = jnp.dot(h, w4_ref[...],
                               preferred_element_type=jnp.float32) + b4_ref[...]


def kernel(x_sv, x_trk, batch_sv, batch_trk,
           sv_w1, sv_b1, sv_w2, sv_b2, trk_w1, trk_b1, trk_w2, trk_b2,
           c1_w, c1_b, c2_w, c2_b,
           o_w1, o_b1, o_w2, o_b2, o_w3, o_b3, o_w4, o_b4):
    c1_top, c1_bot = c1_w[:_HID], c1_w[_HID:]
    c2_top, c2_bot = c2_w[:_HID], c2_w[_HID:]
    wd1 = c1_top - c1_bot
    wd2 = c2_top - c2_bot

    row = lambda v: v.reshape(1, -1)

    sv_enc, y1, sv_n = pl.pallas_call(
        _enc_sv_body,
        grid=(_N_SV // _POOL_BLK,),
        in_specs=[
            pl.BlockSpec((_POOL_BLK, 2), lambda i: (i, 0)),
            pl.BlockSpec((2, _HID), lambda i: (0, 0)),
            pl.BlockSpec((1, _HID), lambda i: (0, 0)),
            pl.BlockSpec((_HID, _HID), lambda i: (0, 0)),
            pl.BlockSpec((1, _HID), lambda i: (0, 0)),
            pl.BlockSpec((_HID, _HID), lambda i: (0, 0)),
        ],
        out_specs=[
            pl.BlockSpec((_POOL_BLK, _HID), lambda i: (i, 0)),
            pl.BlockSpec((_POOL_BLK, _HID), lambda i: (i, 0)),
            pl.BlockSpec((_POOL_BLK, 1), lambda i: (i, 0)),
        ],
        out_shape=[
            jax.ShapeDtypeStruct((_N_SV, _HID), jnp.float32),
            jax.ShapeDtypeStruct((_N_SV, _HID), jnp.float32),
            jax.ShapeDtypeStruct((_N_SV, 1), jnp.float32),
        ],
    )(x_sv, sv_w1, row(sv_b1), sv_w2, row(sv_b2), c1_bot)

    trk_enc, a1, a2, trk_n = pl.pallas_call(
        _enc_trk_body,
        grid=(_N_TRK // _POOL_BLK,),
        in_specs=[
            pl.BlockSpec((_POOL_BLK, 8), lambda i: (i, 0)),
            pl.BlockSpec((8, _HID), lambda i: (0, 0)),
            pl.BlockSpec((1, _HID), lambda i: (0, 0)),
            pl.BlockSpec((_HID, _HID), lambda i: (0, 0)),
            pl.BlockSpec((1, _HID), lambda i: (0, 0)),
            pl.BlockSpec((_HID, _HID), lambda i: (0, 0)),
            pl.BlockSpec((1, _HID), lambda i: (0, 0)),
            pl.BlockSpec((_HID, _HID), lambda i: (0, 0)),
            pl.BlockSpec((1, _HID), lambda i: (0, 0)),
        ],
        out_specs=[
            pl.BlockSpec((_POOL_BLK, _HID), lambda i: (i, 0)),
            pl.BlockSpec((_POOL_BLK, _HID), lambda i: (i, 0)),
            pl.BlockSpec((_POOL_BLK, _HID), lambda i: (i, 0)),
            pl.BlockSpec((_POOL_BLK, 1), lambda i: (i, 0)),
        ],
        out_shape=[
            jax.ShapeDtypeStruct((_N_TRK, _HID), jnp.float32),
            jax.ShapeDtypeStruct((_N_TRK, _HID), jnp.float32),
            jax.ShapeDtypeStruct((_N_TRK, _HID), jnp.float32),
            jax.ShapeDtypeStruct((_N_TRK, 1), jnp.float32),
        ],
    )(x_trk, trk_w1, row(trk_b1), trk_w2, row(trk_b2),
      wd1, row(c1_b), wd2, row(c2_b))

    btrk_rows = batch_trk.reshape(-1, 1, _TILE)
    bsv_col = batch_sv.reshape(-1, 1)
    btrk_col = batch_trk.reshape(-1, 1)
    trkn_rows = trk_n.reshape(-1, 1, _TILE)

    sband1 = _band_bounds(batch_trk, batch_sv)
    idx1 = _banded_knn(sband1, btrk_rows, trk_enc, trkn_rows,
                       sv_enc, sv_n, bsv_col)
    m1 = _gather_max(y1, idx1)

    f1, y2, f1_n = pl.pallas_call(
        _feats1_body,
        grid=(_N_TRK // _POOL_BLK,),
        in_specs=[
            pl.BlockSpec((_POOL_BLK, _HID), lambda i: (i, 0)),
            pl.BlockSpec((_POOL_BLK, _HID), lambda i: (i, 0)),
            pl.BlockSpec((_HID, _HID), lambda i: (0, 0)),
        ],
        out_specs=[
            pl.BlockSpec((_POOL_BLK, _HID), lambda i: (i, 0)),
            pl.BlockSpec((_POOL_BLK, _HID), lambda i: (i, 0)),
            pl.BlockSpec((_POOL_BLK, 1), lambda i: (i, 0)),
        ],
        out_shape=[
            jax.ShapeDtypeStruct((_N_TRK, _HID), jnp.float32),
            jax.ShapeDtypeStruct((_N_TRK, _HID), jnp.float32),
            jax.ShapeDtypeStruct((_N_TRK, 1), jnp.float32),
        ],
    )(a1, m1, c2_bot)

    sband2 = _band_bounds(batch_trk, batch_trk)
    idx2 = _banded_knn(sband2, btrk_rows, trk_enc, trkn_rows,
                       f1, f1_n, btrk_col)
    m2 = _gather_max(y2, idx2)

    out = pl.pallas_call(
        _pool_body,
        grid=(_N_TRK // _POOL_BLK,),
        in_specs=[
            pl.BlockSpec((_POOL_BLK, _HID), lambda i: (i, 0)),
            pl.BlockSpec((_POOL_BLK, _HID), lambda i: (i, 0)),
            pl.BlockSpec((_POOL_BLK, 1), lambda i: (i, 0)),
            pl.BlockSpec((_HID, 64), lambda i: (0, 0)),
            pl.BlockSpec((1, 64), lambda i: (0, 0)),
            pl.BlockSpec((64, 32), lambda i: (0, 0)),
            pl.BlockSpec((1, 32), lambda i: (0, 0)),
            pl.BlockSpec((32, 4), lambda i: (0, 0)),
            pl.BlockSpec((1, 4), lambda i: (0, 0)),
            pl.BlockSpec((4, 1), lambda i: (0, 0)),
            pl.BlockSpec((1, 1), lambda i: (0, 0)),
        ],
        out_specs=pl.BlockSpec((_B, 1), lambda i: (0, 0)),
        out_shape=jax.ShapeDtypeStruct((_B, 1), jnp.float32),
        scratch_shapes=[
            pltpu.VMEM((_B, _HID), jnp.float32),
            pltpu.VMEM((1, _B), jnp.float32),
        ],
    )(a2, m2, btrk_col, o_w1, row(o_b1), o_w2, row(o_b2),
      o_w3, row(o_b3), o_w4, row(o_b4))

    return (out, jnp.arange(_B, dtype=batch_trk.dtype))
